# baseline (speedup 1.0000x reference)
import functools

import jax
import jax.numpy as jnp
from jax import lax
from jax.experimental import pallas as pl
from jax.experimental.pallas import tpu as pltpu
from jax.experimental.pallas import tpu_sc as plsc

_NC = 2
_NS = 16
_NW = _NC * _NS


@jax.jit
def _sc_gather(x, table):
    Bx, S = x.shape
    V, D = table.shape
    B = Bx * S
    b_per_w = B // _NW
    C = 128
    n_chunks = b_per_w // C
    w_per_row = S // b_per_w

    mesh = plsc.VectorSubcoreMesh(core_axis_name="c", subcore_axis_name="s")

    @functools.partial(
        pl.kernel,
        mesh=mesh,
        out_type=jax.ShapeDtypeStruct((B, D), jnp.float32),
        scratch_types=[
            pltpu.VMEM((b_per_w,), jnp.int32),
            pltpu.VMEM((C, D), jnp.float32),
            pltpu.SemaphoreType.DMA,
        ],
    )
    def k(x_hbm, table_hbm, out_hbm, idx_v, rows_v, sem):
        wid = lax.axis_index("s") * _NC + lax.axis_index("c")
        base = wid * b_per_w
        r = wid // w_per_row
        col = (wid % w_per_row) * b_per_w
        pltpu.sync_copy(x_hbm.at[r, pl.ds(col, b_per_w)], idx_v)
        for g in range(n_chunks):
            pltpu.async_copy(
                table_hbm.at[idx_v.at[pl.ds(g * C, C)]], rows_v, sem
            ).wait()
            plsc.subcore_barrier()
            pltpu.sync_copy(rows_v, out_hbm.at[pl.ds(base + g * C, C)])
            plsc.subcore_barrier()

    return k(x, table)


def kernel(x, W_E):
    B, S = x.shape
    V, D = W_E.shape
    out = _sc_gather(x.astype(jnp.int32), W_E)
    return out.reshape(B, S, D)
